# Initial kernel scaffold; baseline (speedup 1.0000x reference)
import jax
import jax.numpy as jnp
from jax.experimental import pallas as pl


def kernel(x, w1, b1, w2, b2, wfc1, bfc1, l0w, l0b, l1w, l1b, l2w, l2b):
    raise NotImplementedError("write your pallas kernel here")


# speedup vs baseline: 1.0954x; 1.0954x over previous
import jax
import jax.numpy as jnp
from jax.experimental import pallas as pl
from jax.experimental.pallas import tpu as pltpu


def _conv_stack_kernel(p_ref, w1_ref, b1_ref, w2_ref, b2_ref, o_ref):
    bt = o_ref.shape[0]
    p = p_ref[...].reshape(bt * 784, 9)
    a = jnp.dot(p, w1_ref[...], preferred_element_type=jnp.float32)
    a = jnp.maximum(a + b1_ref[...], 0.0)
    a = jnp.max(a.reshape(bt * 392, 2, 16), axis=1)
    a = jnp.max(a.reshape(bt, 14, 2, 14, 16), axis=2)
    zh = jnp.zeros((bt, 1, 14, 16), jnp.float32)
    ap = jnp.concatenate([zh, a, zh], axis=1)
    zw = jnp.zeros((bt, 16, 1, 16), jnp.float32)
    ap = jnp.concatenate([zw, ap, zw], axis=2)
    taps = [ap[:, dy:dy + 14, dx:dx + 14, :]
            for dy in range(3) for dx in range(3)]
    pat = jnp.concatenate(taps, axis=-1).reshape(bt * 196, 144)
    c = jnp.dot(pat, w2_ref[...], preferred_element_type=jnp.float32)
    c = jnp.maximum(c + b2_ref[...], 0.0)
    c = jnp.max(c.reshape(bt * 98, 2, 32), axis=1)
    c = jnp.max(c.reshape(bt, 7, 2, 7, 32), axis=2)
    o_ref[...] = c.reshape(bt, 49, 32)


def _fc_logsoftmax_kernel(h_ref, wc_ref, bc_ref, o_ref):
    z = jnp.dot(h_ref[...], wc_ref[...],
                preferred_element_type=jnp.float32) + bc_ref[...]
    m = jnp.max(z, axis=-1, keepdims=True)
    s = z - m
    lse = jnp.log(jnp.sum(jnp.exp(s), axis=-1, keepdims=True))
    o_ref[...] = s - lse


def kernel(x, w1, b1, w2, b2, wfc1, bfc1, l0w, l0b, l1w, l1b, l2w, l2b):
    B = x.shape[0]
    bt1 = 32 if B % 32 == 0 else B
    bt2 = 1024 if B % 1024 == 0 else B

    x2 = x.reshape(B, 28, 28)
    xpad = jnp.pad(x2, ((0, 0), (1, 1), (1, 1)))
    p1 = jnp.stack([xpad[:, dy:dy + 28, dx:dx + 28]
                    for dy in range(3) for dx in range(3)],
                   axis=-1).reshape(B, 784, 9)
    w2r = w2.reshape(144, 32)

    t1 = l1w @ l2w
    t0 = l0w @ t1
    wc = wfc1 @ t0
    bc = bfc1 @ t0 + l0b @ t1 + l1b @ l2w + l2b

    conv_out = pl.pallas_call(
        _conv_stack_kernel,
        out_shape=jax.ShapeDtypeStruct((B, 49, 32), jnp.float32),
        grid=(B // bt1,),
        in_specs=[
            pl.BlockSpec((bt1, 784, 9), lambda b: (b, 0, 0)),
            pl.BlockSpec((9, 16), lambda b: (0, 0)),
            pl.BlockSpec((1, 16), lambda b: (0, 0)),
            pl.BlockSpec((144, 32), lambda b: (0, 0)),
            pl.BlockSpec((1, 32), lambda b: (0, 0)),
        ],
        out_specs=pl.BlockSpec((bt1, 49, 32), lambda b: (b, 0, 0)),
        compiler_params=pltpu.CompilerParams(
            dimension_semantics=("parallel",)),
        cost_estimate=pl.CostEstimate(
            flops=2 * B * (784 * 9 * 16 + 196 * 144 * 32),
            transcendentals=0,
            bytes_accessed=4 * (B * 784 * 9 + B * 49 * 32)),
    )(p1, w1, b1, w2r, b2)

    h = conv_out.reshape(B, 1568)
    out = pl.pallas_call(
        _fc_logsoftmax_kernel,
        out_shape=jax.ShapeDtypeStruct((B, 10), jnp.float32),
        grid=(B // bt2,),
        in_specs=[
            pl.BlockSpec((bt2, 1568), lambda b: (b, 0)),
            pl.BlockSpec((1568, 10), lambda b: (0, 0)),
            pl.BlockSpec((1, 10), lambda b: (0, 0)),
        ],
        out_specs=pl.BlockSpec((bt2, 10), lambda b: (b, 0)),
        compiler_params=pltpu.CompilerParams(
            dimension_semantics=("parallel",)),
        cost_estimate=pl.CostEstimate(
            flops=2 * B * 1568 * 10,
            transcendentals=B * 10,
            bytes_accessed=4 * (B * 1568 + B * 10)),
    )(h, wc, bc)
    return out


# speedup vs baseline: 74.0313x; 67.5828x over previous
import numpy as np

import jax
import jax.numpy as jnp
from jax.experimental import pallas as pl
from jax.experimental.pallas import tpu as pltpu


def _conv_stack_kernel(x_ref, m1_ref, b1t_ref, m2_ref, b2t_ref, o_ref):
    bt = o_ref.shape[0]
    f32 = jnp.float32
    xv = x_ref[...]
    zc = jnp.zeros((bt, 28, 1), f32)
    xv = jnp.concatenate([zc, xv, zc], axis=2)
    zr = jnp.zeros((bt, 1, 30), f32)
    xv = jnp.concatenate([zr, xv, zr], axis=1)
    y = jnp.dot(xv.reshape(bt * 30, 30), m1_ref[...],
                preferred_element_type=f32)
    y = y.reshape(bt, 30, 1536)
    a = (y[:, 0:28, 0:512] + y[:, 1:29, 512:1024] + y[:, 2:30, 1024:1536])
    a = jnp.maximum(a + b1t_ref[...], 0.0)
    a = jnp.maximum(a[:, :, 0:256], a[:, :, 256:512])
    a = jnp.max(a.reshape(bt, 14, 2, 256), axis=2)
    z16 = jnp.zeros((bt, 14, 16), f32)
    a = jnp.concatenate([z16, a[:, :, 0:224], z16], axis=2)
    zr2 = jnp.zeros((bt, 1, 256), f32)
    a = jnp.concatenate([zr2, a, zr2], axis=1)
    y2 = jnp.dot(a.reshape(bt * 16, 256), m2_ref[...],
                 preferred_element_type=f32)
    y2 = y2.reshape(bt, 16, 1536)
    c = (y2[:, 0:14, 0:512] + y2[:, 1:15, 512:1024] + y2[:, 2:16, 1024:1536])
    c = jnp.maximum(c + b2t_ref[...], 0.0)
    c = jnp.maximum(c[:, :, 0:256], c[:, :, 256:512])
    c = jnp.max(c.reshape(bt, 7, 2, 256), axis=2)
    o_ref[...] = c


def _fc_logsoftmax_kernel(h_ref, wc_ref, bc_ref, o_ref):
    z = jnp.dot(h_ref[...], wc_ref[...],
                preferred_element_type=jnp.float32) + bc_ref[...]
    m = jnp.max(z, axis=-1, keepdims=True)
    s = z - m
    lse = jnp.log(jnp.sum(jnp.exp(s), axis=-1, keepdims=True))
    o_ref[...] = s - lse


def _banded(taps, n_in, n_out, blk):
    cin, cout = taps[0].shape
    n_w = n_out // 2 // cout
    n_slots = n_in // cin
    blocks = []
    for dy in range(3):
        for par in range(2):
            m = jnp.zeros((n_in, n_w * cout), jnp.float32)
            for dx in range(3):
                s = np.zeros((n_slots, n_w), np.float32)
                for w in range(n_w):
                    s[2 * w + par + dx, w] = 1.0
                m = m + jnp.kron(jnp.asarray(s), taps[dy * 3 + dx])
            pad = jnp.zeros((n_in, 256 - n_w * cout), jnp.float32)
            blocks.append(jnp.concatenate([m, pad], axis=1))
    return jnp.concatenate(blocks, axis=1)


def kernel(x, w1, b1, w2, b2, wfc1, bfc1, l0w, l0b, l1w, l1b, l2w, l2b):
    B = x.shape[0]
    bt1 = 64 if B % 64 == 0 else B
    bt2 = 1024 if B % 1024 == 0 else B

    w1taps = [w1[t].reshape(1, 16) for t in range(9)]
    m1 = _banded(w1taps, 30, 448, 16)
    m2 = _banded([w2[t] for t in range(9)], 256, 448, 16)
    b1t = jnp.tile(b1, (1, 32)).reshape(1, 1, 512)
    b2t = jnp.tile(b2, (1, 16)).reshape(1, 1, 512)

    t1 = l1w @ l2w
    t0 = l0w @ t1
    wc = wfc1 @ t0
    bc = bfc1 @ t0 + l0b @ t1 + l1b @ l2w + l2b
    wc2 = jnp.pad(wc.reshape(7, 7 * 32, 10),
                  ((0, 0), (0, 32), (0, 0))).reshape(1792, 10)

    conv_out = pl.pallas_call(
        _conv_stack_kernel,
        out_shape=jax.ShapeDtypeStruct((B, 7, 256), jnp.float32),
        grid=(B // bt1,),
        in_specs=[
            pl.BlockSpec((bt1, 28, 28), lambda b: (b, 0, 0)),
            pl.BlockSpec((30, 1536), lambda b: (0, 0)),
            pl.BlockSpec((1, 1, 512), lambda b: (0, 0, 0)),
            pl.BlockSpec((256, 1536), lambda b: (0, 0)),
            pl.BlockSpec((1, 1, 512), lambda b: (0, 0, 0)),
        ],
        out_specs=pl.BlockSpec((bt1, 7, 256), lambda b: (b, 0, 0)),
        compiler_params=pltpu.CompilerParams(
            dimension_semantics=("parallel",)),
        cost_estimate=pl.CostEstimate(
            flops=2 * B * (30 * 30 * 1536 + 16 * 256 * 1536),
            transcendentals=0,
            bytes_accessed=4 * (B * 784 + B * 7 * 256)),
    )(x.reshape(B, 28, 28), m1, b1t, m2, b2t)

    h = conv_out.reshape(B, 1792)
    out = pl.pallas_call(
        _fc_logsoftmax_kernel,
        out_shape=jax.ShapeDtypeStruct((B, 10), jnp.float32),
        grid=(B // bt2,),
        in_specs=[
            pl.BlockSpec((bt2, 1792), lambda b: (b, 0)),
            pl.BlockSpec((1792, 10), lambda b: (0, 0)),
            pl.BlockSpec((1, 10), lambda b: (0, 0)),
        ],
        out_specs=pl.BlockSpec((bt2, 10), lambda b: (b, 0)),
        compiler_params=pltpu.CompilerParams(
            dimension_semantics=("parallel",)),
        cost_estimate=pl.CostEstimate(
            flops=2 * B * 1792 * 10,
            transcendentals=B * 10,
            bytes_accessed=4 * (B * 1792 + B * 10)),
    )(h, wc2, bc)
    return out


# speedup vs baseline: 99.2716x; 1.3409x over previous
import numpy as np

import jax
import jax.numpy as jnp
from jax.experimental import pallas as pl
from jax.experimental.pallas import tpu as pltpu


def _conv_stack_kernel(x_ref, m1_ref, b1t_ref, m2_ref, b2t_ref, o_ref):
    bt = o_ref.shape[0]
    f32 = jnp.float32
    xv = x_ref[...]
    zr = jnp.zeros((bt, 1, 28), f32)
    xp = jnp.concatenate([zr, xv, zr], axis=1)
    lhs = jnp.concatenate([xp[:, 0:28, :], xp[:, 1:29, :], xp[:, 2:30, :]],
                          axis=2)
    a = jnp.dot(lhs.reshape(bt * 28, 84), m1_ref[...],
                preferred_element_type=f32)
    a = a.reshape(bt, 28, 512)
    a = jnp.maximum(a + b1t_ref[...], 0.0)
    a = jnp.maximum(a[:, :, 0:256], a[:, :, 256:512])
    a = jnp.max(a.reshape(bt, 14, 2, 256), axis=2)
    a = a[:, :, 0:224]
    zr2 = jnp.zeros((bt, 1, 224), f32)
    a = jnp.concatenate([zr2, a, zr2], axis=1)
    lhs2 = jnp.concatenate([a[:, 0:14, :], a[:, 1:15, :], a[:, 2:16, :]],
                           axis=2)
    c = jnp.dot(lhs2.reshape(bt * 14, 672), m2_ref[...],
                preferred_element_type=f32)
    c = c.reshape(bt, 14, 512)
    c = jnp.maximum(c + b2t_ref[...], 0.0)
    c = jnp.maximum(c[:, :, 0:256], c[:, :, 256:512])
    c = jnp.max(c.reshape(bt, 7, 2, 256), axis=2)
    o_ref[...] = c


def _fc_logsoftmax_kernel(h_ref, wc_ref, bc_ref, o_ref):
    z = jnp.dot(h_ref[...], wc_ref[...],
                preferred_element_type=jnp.float32) + bc_ref[...]
    m = jnp.max(z, axis=-1, keepdims=True)
    s = z - m
    lse = jnp.log(jnp.sum(jnp.exp(s), axis=-1, keepdims=True))
    o_ref[...] = s - lse


def _banded(taps, n_slots, n_w, cin, cout):
    dy_blocks = []
    for dy in range(3):
        par_blocks = []
        for par in range(2):
            m = jnp.zeros((n_slots * cin, n_w * cout), jnp.float32)
            for dx in range(3):
                s = np.zeros((n_slots, n_w), np.float32)
                for w4 in range(n_w):
                    win = 2 * w4 + par + dx - 1
                    if 0 <= win < n_slots:
                        s[win, w4] = 1.0
                m = m + jnp.kron(jnp.asarray(s), taps[dy * 3 + dx])
            pad = jnp.zeros((n_slots * cin, 256 - n_w * cout), jnp.float32)
            par_blocks.append(jnp.concatenate([m, pad], axis=1))
        dy_blocks.append(jnp.concatenate(par_blocks, axis=1))
    return jnp.concatenate(dy_blocks, axis=0)


def kernel(x, w1, b1, w2, b2, wfc1, bfc1, l0w, l0b, l1w, l1b, l2w, l2b):
    B = x.shape[0]
    bt1 = 128 if B % 128 == 0 else B
    bt2 = 1024 if B % 1024 == 0 else B

    w1taps = [w1[t].reshape(1, 16) for t in range(9)]
    m1 = _banded(w1taps, 28, 14, 1, 16)
    m2 = _banded([w2[t] for t in range(9)], 14, 7, 16, 32)
    b1t = jnp.tile(b1, (1, 32)).reshape(1, 1, 512)
    b2t = jnp.tile(b2, (1, 16)).reshape(1, 1, 512)

    t1 = l1w @ l2w
    t0 = l0w @ t1
    wc = wfc1 @ t0
    bc = bfc1 @ t0 + l0b @ t1 + l1b @ l2w + l2b
    wc2 = jnp.pad(wc.reshape(7, 7 * 32, 10),
                  ((0, 0), (0, 32), (0, 0))).reshape(1792, 10)

    conv_out = pl.pallas_call(
        _conv_stack_kernel,
        out_shape=jax.ShapeDtypeStruct((B, 7, 256), jnp.float32),
        grid=(B // bt1,),
        in_specs=[
            pl.BlockSpec((bt1, 28, 28), lambda b: (b, 0, 0)),
            pl.BlockSpec((84, 512), lambda b: (0, 0)),
            pl.BlockSpec((1, 1, 512), lambda b: (0, 0, 0)),
            pl.BlockSpec((672, 512), lambda b: (0, 0)),
            pl.BlockSpec((1, 1, 512), lambda b: (0, 0, 0)),
        ],
        out_specs=pl.BlockSpec((bt1, 7, 256), lambda b: (b, 0, 0)),
        compiler_params=pltpu.CompilerParams(
            dimension_semantics=("parallel",)),
        cost_estimate=pl.CostEstimate(
            flops=2 * B * (28 * 84 * 512 + 14 * 672 * 512),
            transcendentals=0,
            bytes_accessed=4 * (B * 784 + B * 7 * 256)),
    )(x.reshape(B, 28, 28), m1, b1t, m2, b2t)

    h = conv_out.reshape(B, 1792)
    out = pl.pallas_call(
        _fc_logsoftmax_kernel,
        out_shape=jax.ShapeDtypeStruct((B, 10), jnp.float32),
        grid=(B // bt2,),
        in_specs=[
            pl.BlockSpec((bt2, 1792), lambda b: (b, 0)),
            pl.BlockSpec((1792, 10), lambda b: (0, 0)),
            pl.BlockSpec((1, 10), lambda b: (0, 0)),
        ],
        out_specs=pl.BlockSpec((bt2, 10), lambda b: (b, 0)),
        compiler_params=pltpu.CompilerParams(
            dimension_semantics=("parallel",)),
        cost_estimate=pl.CostEstimate(
            flops=2 * B * 1792 * 10,
            transcendentals=B * 10,
            bytes_accessed=4 * (B * 1792 + B * 10)),
    )(h, wc2, bc)
    return out


# speedup vs baseline: 102.0172x; 1.0277x over previous
import numpy as np

import jax
import jax.numpy as jnp
from jax.experimental import pallas as pl
from jax.experimental.pallas import tpu as pltpu


def _conv_stack_kernel(x_ref, m1_ref, b1t_ref, m2_ref, b2t_ref, mask_ref,
                       o_ref):
    bt = o_ref.shape[0]
    f32 = jnp.float32
    xv = x_ref[...]
    z1 = jnp.zeros((bt, 1, 28), f32)
    z5 = jnp.zeros((bt, 5, 28), f32)
    xp = jnp.concatenate([z1, xv, z5], axis=1)
    lhs = jnp.concatenate([xp[:, 0:32, :], xp[:, 1:33, :], xp[:, 2:34, :]],
                          axis=2)
    a = jnp.dot(lhs.reshape(bt * 32, 84), m1_ref[...],
                preferred_element_type=f32)
    a = jnp.maximum(a + b1t_ref[...], 0.0)
    a = jnp.maximum(a[:, 0:256], a[:, 256:512])
    a = jnp.max(a.reshape(bt * 16, 2, 256), axis=1)
    a = a.reshape(bt, 16, 256) * mask_ref[...]
    zr = jnp.zeros((bt, 1, 256), f32)
    hp = jnp.concatenate([zr, a, zr], axis=1)
    lhs2 = jnp.concatenate([hp[:, 0:16, :], hp[:, 1:17, :], hp[:, 2:18, :]],
                           axis=2)
    c = jnp.dot(lhs2.reshape(bt * 16, 768), m2_ref[...],
                preferred_element_type=f32)
    c = jnp.maximum(c + b2t_ref[...], 0.0)
    c = jnp.maximum(c[:, 0:256], c[:, 256:512])
    c = jnp.max(c.reshape(bt * 8, 2, 256), axis=1)
    o_ref[...] = c.reshape(bt, 8, 256)


def _fc_logsoftmax_kernel(h_ref, wc_ref, bc_ref, o_ref):
    z = jnp.dot(h_ref[...], wc_ref[...],
                preferred_element_type=jnp.float32) + bc_ref[...]
    m = jnp.max(z, axis=-1, keepdims=True)
    s = z - m
    lse = jnp.log(jnp.sum(jnp.exp(s), axis=-1, keepdims=True))
    o_ref[...] = s - lse


def _banded(taps, n_slots, n_w, cin, cout):
    dy_blocks = []
    for dy in range(3):
        par_blocks = []
        for par in range(2):
            m = jnp.zeros((n_slots * cin, n_w * cout), jnp.float32)
            for dx in range(3):
                s = np.zeros((n_slots, n_w), np.float32)
                for w4 in range(n_w):
                    win = 2 * w4 + par + dx - 1
                    if 0 <= win < n_slots:
                        s[win, w4] = 1.0
                m = m + jnp.kron(jnp.asarray(s), taps[dy * 3 + dx])
            pad = jnp.zeros((n_slots * cin, 256 - n_w * cout), jnp.float32)
            par_blocks.append(jnp.concatenate([m, pad], axis=1))
        dy_blocks.append(jnp.concatenate(par_blocks, axis=1))
    return jnp.concatenate(dy_blocks, axis=0)


def kernel(x, w1, b1, w2, b2, wfc1, bfc1, l0w, l0b, l1w, l1b, l2w, l2b):
    B = x.shape[0]
    bt1 = 128 if B % 128 == 0 else B
    bt2 = 1024 if B % 1024 == 0 else B

    w1taps = [w1[t].reshape(1, 16) for t in range(9)]
    m1 = _banded(w1taps, 28, 14, 1, 16)
    m2 = _banded([w2[t] for t in range(9)], 16, 7, 16, 32)
    b1t = jnp.tile(b1, (1, 32))
    b2t = jnp.tile(b2, (1, 16))
    mask = np.zeros((1, 16, 256), np.float32)
    mask[:, 0:14, 0:224] = 1.0
    mask = jnp.asarray(mask)

    t1 = l1w @ l2w
    t0 = l0w @ t1
    wc = wfc1 @ t0
    bc = bfc1 @ t0 + l0b @ t1 + l1b @ l2w + l2b
    wc2 = jnp.pad(wc.reshape(7, 7 * 32, 10),
                  ((0, 1), (0, 32), (0, 0))).reshape(2048, 10)

    conv_out = pl.pallas_call(
        _conv_stack_kernel,
        out_shape=jax.ShapeDtypeStruct((B, 8, 256), jnp.float32),
        grid=(B // bt1,),
        in_specs=[
            pl.BlockSpec((bt1, 28, 28), lambda b: (b, 0, 0)),
            pl.BlockSpec((84, 512), lambda b: (0, 0)),
            pl.BlockSpec((1, 512), lambda b: (0, 0)),
            pl.BlockSpec((768, 512), lambda b: (0, 0)),
            pl.BlockSpec((1, 512), lambda b: (0, 0)),
            pl.BlockSpec((1, 16, 256), lambda b: (0, 0, 0)),
        ],
        out_specs=pl.BlockSpec((bt1, 8, 256), lambda b: (b, 0, 0)),
        compiler_params=pltpu.CompilerParams(
            dimension_semantics=("parallel",)),
        cost_estimate=pl.CostEstimate(
            flops=2 * B * (32 * 84 * 512 + 16 * 768 * 512),
            transcendentals=0,
            bytes_accessed=4 * (B * 784 + B * 8 * 256)),
    )(x.reshape(B, 28, 28), m1, b1t, m2, b2t, mask)

    h = conv_out.reshape(B, 2048)
    out = pl.pallas_call(
        _fc_logsoftmax_kernel,
        out_shape=jax.ShapeDtypeStruct((B, 10), jnp.float32),
        grid=(B // bt2,),
        in_specs=[
            pl.BlockSpec((bt2, 2048), lambda b: (b, 0)),
            pl.BlockSpec((2048, 10), lambda b: (0, 0)),
            pl.BlockSpec((1, 10), lambda b: (0, 0)),
        ],
        out_specs=pl.BlockSpec((bt2, 10), lambda b: (b, 0)),
        compiler_params=pltpu.CompilerParams(
            dimension_semantics=("parallel",)),
        cost_estimate=pl.CostEstimate(
            flops=2 * B * 2048 * 10,
            transcendentals=B * 10,
            bytes_accessed=4 * (B * 2048 + B * 10)),
    )(h, wc2, bc)
    return out


# speedup vs baseline: 105.1094x; 1.0303x over previous
import numpy as np

import jax
import jax.numpy as jnp
from jax.experimental import pallas as pl
from jax.experimental.pallas import tpu as pltpu


def _conv_stack_kernel(x_ref, m1_ref, b1t_ref, m2_ref, b2t_ref, mask_ref,
                       o_ref):
    bt = o_ref.shape[0]
    f32 = jnp.float32
    xq = x_ref[...]
    z1 = jnp.zeros((bt, 1, 56), f32)
    z3 = jnp.zeros((bt, 3, 56), f32)
    xqp = jnp.concatenate([z1, xq, z3], axis=1)
    lhs = jnp.concatenate(
        [xqp[:, 0:16, :], xqp[:, 1:17, :], xqp[:, 2:18, :]],
        axis=2)
    a = jnp.dot(lhs.reshape(bt * 16, 168), m1_ref[...],
                preferred_element_type=f32)
    a = jnp.maximum(a + b1t_ref[...], 0.0)
    a = jnp.maximum(jnp.maximum(a[:, 0:256], a[:, 256:512]),
                    jnp.maximum(a[:, 512:768], a[:, 768:1024]))
    a = a.reshape(bt, 16, 256) * mask_ref[...]
    zr = jnp.zeros((bt, 1, 256), f32)
    hp = jnp.concatenate([zr, a, zr], axis=1)
    lhs2 = jnp.concatenate([hp[:, 0:16, :], hp[:, 1:17, :], hp[:, 2:18, :]],
                           axis=2)
    c = jnp.dot(lhs2.reshape(bt * 16, 768), m2_ref[...],
                preferred_element_type=f32)
    c = jnp.maximum(c + b2t_ref[...], 0.0)
    c = jnp.maximum(c[:, 0:256], c[:, 256:512])
    c = jnp.max(c.reshape(bt * 8, 2, 256), axis=1)
    o_ref[...] = c.reshape(bt, 8, 256)


def _fc_logsoftmax_kernel(h_ref, wc_ref, bc_ref, o_ref):
    z = jnp.dot(h_ref[...], wc_ref[...],
                preferred_element_type=jnp.float32) + bc_ref[...]
    m = jnp.max(z, axis=-1, keepdims=True)
    s = z - m
    lse = jnp.log(jnp.sum(jnp.exp(s), axis=-1, keepdims=True))
    o_ref[...] = s - lse


def _m1_paired(taps):
    cols = []
    for hpar in range(2):
        for wpar in range(2):
            m = jnp.zeros((168, 224), jnp.float32)
            for j in range(3):
                for rowpar in range(2):
                    dy = 2 * j + rowpar - 1 - hpar
                    if not 0 <= dy < 3:
                        continue
                    for dx in range(3):
                        s = np.zeros((168, 14), np.float32)
                        for w4 in range(14):
                            win = 2 * w4 + wpar + dx - 1
                            if 0 <= win < 28:
                                s[j * 56 + rowpar * 28 + win, w4] = 1.0
                        m = m + jnp.kron(jnp.asarray(s),
                                         taps[dy * 3 + dx])
            cols.append(jnp.concatenate(
                [m, jnp.zeros((168, 32), jnp.float32)], axis=1))
    return jnp.concatenate(cols, axis=1)


def _banded(taps, n_slots, n_w, cin, cout):
    dy_blocks = []
    for dy in range(3):
        par_blocks = []
        for par in range(2):
            m = jnp.zeros((n_slots * cin, n_w * cout), jnp.float32)
            for dx in range(3):
                s = np.zeros((n_slots, n_w), np.float32)
                for w4 in range(n_w):
                    win = 2 * w4 + par + dx - 1
                    if 0 <= win < n_slots:
                        s[win, w4] = 1.0
                m = m + jnp.kron(jnp.asarray(s), taps[dy * 3 + dx])
            pad = jnp.zeros((n_slots * cin, 256 - n_w * cout), jnp.float32)
            par_blocks.append(jnp.concatenate([m, pad], axis=1))
        dy_blocks.append(jnp.concatenate(par_blocks, axis=1))
    return jnp.concatenate(dy_blocks, axis=0)


def kernel(x, w1, b1, w2, b2, wfc1, bfc1, l0w, l0b, l1w, l1b, l2w, l2b):
    B = x.shape[0]
    bt1 = 128 if B % 128 == 0 else B
    bt2 = 1024 if B % 1024 == 0 else B

    w1taps = [w1[t].reshape(1, 16) for t in range(9)]
    m1 = _m1_paired(w1taps)
    m2 = _banded([w2[t] for t in range(9)], 16, 7, 16, 32)
    b1t = jnp.tile(b1, (1, 64))
    b2t = jnp.tile(b2, (1, 16))
    mask = np.zeros((1, 16, 256), np.float32)
    mask[:, 0:14, 0:224] = 1.0
    mask = jnp.asarray(mask)

    t1 = l1w @ l2w
    t0 = l0w @ t1
    wc = wfc1 @ t0
    bc = bfc1 @ t0 + l0b @ t1 + l1b @ l2w + l2b
    wc2 = jnp.pad(wc.reshape(7, 7 * 32, 10),
                  ((0, 1), (0, 32), (0, 0))).reshape(2048, 10)

    conv_out = pl.pallas_call(
        _conv_stack_kernel,
        out_shape=jax.ShapeDtypeStruct((B, 8, 256), jnp.float32),
        grid=(B // bt1,),
        in_specs=[
            pl.BlockSpec((bt1, 14, 56), lambda b: (b, 0, 0)),
            pl.BlockSpec((168, 1024), lambda b: (0, 0)),
            pl.BlockSpec((1, 1024), lambda b: (0, 0)),
            pl.BlockSpec((768, 512), lambda b: (0, 0)),
            pl.BlockSpec((1, 512), lambda b: (0, 0)),
            pl.BlockSpec((1, 16, 256), lambda b: (0, 0, 0)),
        ],
        out_specs=pl.BlockSpec((bt1, 8, 256), lambda b: (b, 0, 0)),
        compiler_params=pltpu.CompilerParams(
            dimension_semantics=("parallel",)),
        cost_estimate=pl.CostEstimate(
            flops=2 * B * (16 * 168 * 1024 + 16 * 768 * 512),
            transcendentals=0,
            bytes_accessed=4 * (B * 784 + B * 8 * 256)),
    )(x.reshape(B, 14, 56), m1, b1t, m2, b2t, mask)

    h = conv_out.reshape(B, 2048)
    out = pl.pallas_call(
        _fc_logsoftmax_kernel,
        out_shape=jax.ShapeDtypeStruct((B, 10), jnp.float32),
        grid=(B // bt2,),
        in_specs=[
            pl.BlockSpec((bt2, 2048), lambda b: (b, 0)),
            pl.BlockSpec((2048, 10), lambda b: (0, 0)),
            pl.BlockSpec((1, 10), lambda b: (0, 0)),
        ],
        out_specs=pl.BlockSpec((bt2, 10), lambda b: (b, 0)),
        compiler_params=pltpu.CompilerParams(
            dimension_semantics=("parallel",)),
        cost_estimate=pl.CostEstimate(
            flops=2 * B * 2048 * 10,
            transcendentals=B * 10,
            bytes_accessed=4 * (B * 2048 + B * 10)),
    )(h, wc2, bc)
    return out


# speedup vs baseline: 110.8352x; 1.0545x over previous
import numpy as np

import jax
import jax.numpy as jnp
from jax.experimental import pallas as pl
from jax.experimental.pallas import tpu as pltpu


def _conv_stack_kernel(x_ref, m1_ref, b1t_ref, m2_ref, b2t_ref, mask_ref,
                       o_ref):
    bt = o_ref.shape[0]
    f32 = jnp.float32
    bf16 = jnp.bfloat16
    xq = x_ref[...]
    z1 = jnp.zeros((bt, 1, 56), bf16)
    z3 = jnp.zeros((bt, 3, 56), bf16)
    xqp = jnp.concatenate([z1, xq, z3], axis=1)
    lhs = jnp.concatenate(
        [xqp[:, 0:16, :], xqp[:, 1:17, :], xqp[:, 2:18, :]],
        axis=2)
    a = jnp.dot(lhs.reshape(bt * 16, 168), m1_ref[...],
                preferred_element_type=f32)
    a = jnp.maximum(a + b1t_ref[...], 0.0)
    a = jnp.maximum(jnp.maximum(a[:, 0:256], a[:, 256:512]),
                    jnp.maximum(a[:, 512:768], a[:, 768:1024]))
    a = (a.reshape(bt, 16, 256) * mask_ref[...]).astype(bf16)
    zr = jnp.zeros((bt, 1, 256), bf16)
    hp = jnp.concatenate([zr, a, zr], axis=1)
    lhs2 = jnp.concatenate([hp[:, 0:16, :], hp[:, 1:17, :], hp[:, 2:18, :]],
                           axis=2)
    c = jnp.dot(lhs2.reshape(bt * 16, 768), m2_ref[...],
                preferred_element_type=f32)
    c = jnp.maximum(c + b2t_ref[...], 0.0)
    c = jnp.maximum(c[:, 0:256], c[:, 256:512])
    c = jnp.max(c.reshape(bt * 8, 2, 256), axis=1)
    o_ref[...] = c.reshape(bt, 8, 256)


def _fc_logsoftmax_kernel(h_ref, wc_ref, bc_ref, o_ref):
    z = jnp.dot(h_ref[...], wc_ref[...],
                preferred_element_type=jnp.float32) + bc_ref[...]
    m = jnp.max(z, axis=-1, keepdims=True)
    s = z - m
    lse = jnp.log(jnp.sum(jnp.exp(s), axis=-1, keepdims=True))
    o_ref[...] = s - lse


def _m1_paired(taps):
    cols = []
    for hpar in range(2):
        for wpar in range(2):
            m = jnp.zeros((168, 224), jnp.float32)
            for j in range(3):
                for rowpar in range(2):
                    dy = 2 * j + rowpar - 1 - hpar
                    if not 0 <= dy < 3:
                        continue
                    for dx in range(3):
                        s = np.zeros((168, 14), np.float32)
                        for w4 in range(14):
                            win = 2 * w4 + wpar + dx - 1
                            if 0 <= win < 28:
                                s[j * 56 + rowpar * 28 + win, w4] = 1.0
                        m = m + jnp.kron(jnp.asarray(s),
                                         taps[dy * 3 + dx])
            cols.append(jnp.concatenate(
                [m, jnp.zeros((168, 32), jnp.float32)], axis=1))
    return jnp.concatenate(cols, axis=1)


def _banded(taps, n_slots, n_w, cin, cout):
    dy_blocks = []
    for dy in range(3):
        par_blocks = []
        for par in range(2):
            m = jnp.zeros((n_slots * cin, n_w * cout), jnp.float32)
            for dx in range(3):
                s = np.zeros((n_slots, n_w), np.float32)
                for w4 in range(n_w):
                    win = 2 * w4 + par + dx - 1
                    if 0 <= win < n_slots:
                        s[win, w4] = 1.0
                m = m + jnp.kron(jnp.asarray(s), taps[dy * 3 + dx])
            pad = jnp.zeros((n_slots * cin, 256 - n_w * cout), jnp.float32)
            par_blocks.append(jnp.concatenate([m, pad], axis=1))
        dy_blocks.append(jnp.concatenate(par_blocks, axis=1))
    return jnp.concatenate(dy_blocks, axis=0)


def kernel(x, w1, b1, w2, b2, wfc1, bfc1, l0w, l0b, l1w, l1b, l2w, l2b):
    B = x.shape[0]
    bt1 = 128 if B % 128 == 0 else B
    bt2 = 1024 if B % 1024 == 0 else B

    w1taps = [w1[t].reshape(1, 16) for t in range(9)]
    m1 = _m1_paired(w1taps).astype(jnp.bfloat16)
    m2 = _banded([w2[t] for t in range(9)],
                 16, 7, 16, 32).astype(jnp.bfloat16)
    b1t = jnp.tile(b1, (1, 64))
    b2t = jnp.tile(b2, (1, 16))
    mask = np.zeros((1, 16, 256), np.float32)
    mask[:, 0:14, 0:224] = 1.0
    mask = jnp.asarray(mask)

    t1 = l1w @ l2w
    t0 = l0w @ t1
    wc = wfc1 @ t0
    bc = bfc1 @ t0 + l0b @ t1 + l1b @ l2w + l2b
    wc2 = jnp.pad(wc.reshape(7, 7 * 32, 10),
                  ((0, 1), (0, 32), (0, 0))).reshape(2048, 10)

    conv_out = pl.pallas_call(
        _conv_stack_kernel,
        out_shape=jax.ShapeDtypeStruct((B, 8, 256), jnp.float32),
        grid=(B // bt1,),
        in_specs=[
            pl.BlockSpec((bt1, 14, 56), lambda b: (b, 0, 0)),
            pl.BlockSpec((168, 1024), lambda b: (0, 0)),
            pl.BlockSpec((1, 1024), lambda b: (0, 0)),
            pl.BlockSpec((768, 512), lambda b: (0, 0)),
            pl.BlockSpec((1, 512), lambda b: (0, 0)),
            pl.BlockSpec((1, 16, 256), lambda b: (0, 0, 0)),
        ],
        out_specs=pl.BlockSpec((bt1, 8, 256), lambda b: (b, 0, 0)),
        compiler_params=pltpu.CompilerParams(
            dimension_semantics=("parallel",)),
        cost_estimate=pl.CostEstimate(
            flops=2 * B * (16 * 168 * 1024 + 16 * 768 * 512),
            transcendentals=0,
            bytes_accessed=4 * (B * 784 + B * 8 * 256)),
    )(x.astype(jnp.bfloat16).reshape(B, 14, 56), m1, b1t, m2, b2t, mask)

    h = conv_out.reshape(B, 2048)
    out = pl.pallas_call(
        _fc_logsoftmax_kernel,
        out_shape=jax.ShapeDtypeStruct((B, 10), jnp.float32),
        grid=(B // bt2,),
        in_specs=[
            pl.BlockSpec((bt2, 2048), lambda b: (b, 0)),
            pl.BlockSpec((2048, 10), lambda b: (0, 0)),
            pl.BlockSpec((1, 10), lambda b: (0, 0)),
        ],
        out_specs=pl.BlockSpec((bt2, 10), lambda b: (b, 0)),
        compiler_params=pltpu.CompilerParams(
            dimension_semantics=("parallel",)),
        cost_estimate=pl.CostEstimate(
            flops=2 * B * 2048 * 10,
            transcendentals=B * 10,
            bytes_accessed=4 * (B * 2048 + B * 10)),
    )(h, wc2, bc)
    return out
